# baseline (speedup 1.0000x reference)
import functools

import jax
import jax.numpy as jnp
from jax import lax
from jax.experimental import pallas as pl
from jax.experimental.pallas import tpu as pltpu
from jax.experimental.pallas import tpu_sc as plsc

N_NODES = 10000
N_PAD = 10240
N_EDGES = 320000
D = 128
L = 16

NC = 2
NS = 16
NW = NC * NS

EPT = N_EDGES // NW
C = 40
NK = EPT // C
ROWS_PER_SUB = N_PAD // NS
OUT_CHUNK = 32

_mesh = plsc.VectorSubcoreMesh(core_axis_name="c", subcore_axis_name="s")


@functools.partial(
    pl.kernel,
    out_type=jax.ShapeDtypeStruct((NC, N_PAD, D), jnp.float32),
    mesh=_mesh,
    scratch_types=[
        pltpu.VMEM_SHARED((N_PAD, D), jnp.float32),
        pltpu.VMEM((EPT,), jnp.int32),
        pltpu.VMEM((C,), jnp.int32),
        pltpu.VMEM((C,), jnp.int32),
        pltpu.VMEM((C,), jnp.int32),
        pltpu.VMEM((C, D), jnp.float32),
        pltpu.VMEM((C, D), jnp.float32),
        pltpu.VMEM((C, D), jnp.float32),
        pltpu.VMEM((C, D), jnp.float32),
        pltpu.VMEM((C, D), jnp.float32),
        pltpu.VMEM((C, D), jnp.float32),
        pltpu.VMEM((OUT_CHUNK, D), jnp.float32),
        pltpu.SemaphoreType.DMA,
        pltpu.SemaphoreType.DMA,
        pltpu.SemaphoreType.DMA,
        pltpu.SemaphoreType.DMA,
        pltpu.SemaphoreType.DMA,
        pltpu.SemaphoreType.DMA,
        pltpu.SemaphoreType.DMA,
        pltpu.SemaphoreType.DMA,
        pltpu.SemaphoreType.DMA,
        pltpu.SemaphoreType.DMA,
        pltpu.SemaphoreType.DMA,
        pltpu.SemaphoreType.DMA,
    ],
)
def _sc_scatter(node_hbm, sidx_hbm, didx_hbm, edge_hbm, out_hbm,
                acc, didx, si0, si1, si2, eb0, eb1, eb2, rw0, rw1, rw2, obuf,
                smi0, smi1, smi2, sme0, sme1, sme2,
                smg0, smg1, smg2, sms0, sms1, sms2):
    c = lax.axis_index("c")
    s = lax.axis_index("s")
    wid = s * NC + c
    sidx = (si0, si1, si2)
    ebuf = (eb0, eb1, eb2)
    rows = (rw0, rw1, rw2)
    sem_i = (smi0, smi1, smi2)
    sem_e = (sme0, sme1, sme2)
    sem_g = (smg0, smg1, smg2)
    sem_s = (sms0, sms1, sms2)

    pltpu.sync_copy(didx_hbm.at[pl.ds(wid * EPT, EPT)], didx)

    zeros = jnp.zeros((L,), jnp.float32)

    def _zrow(i, _):
        for jj in range(D // L):
            obuf[i, pl.ds(jj * L, L)] = zeros
        return 0

    lax.fori_loop(0, OUT_CHUNK, _zrow, 0)
    zcps = [pltpu.make_async_copy(
        obuf, acc.at[pl.ds(s * ROWS_PER_SUB + p * OUT_CHUNK, OUT_CHUNK), :],
        smi0) for p in range(ROWS_PER_SUB // OUT_CHUNK)]
    for cp in zcps:
        cp.start()
    for cp in zcps:
        cp.wait()
    plsc.subcore_barrier()

    def _idx_cp(t, m):
        return pltpu.make_async_copy(
            sidx_hbm.at[pl.ds(wid * EPT + t * C, C)], sidx[m], sem_i[m])

    def _edge_cp(t, m):
        return pltpu.make_async_copy(
            edge_hbm.at[pl.ds(wid * EPT + t * C, C), :], ebuf[m], sem_e[m])

    def _gather_cp(m):
        return pltpu.make_async_copy(
            node_hbm.at[sidx[m]], rows[m], sem_g[m])

    def _scatter_cp(t, m):
        return pltpu.make_async_copy(
            ebuf[m], acc.at[didx.at[pl.ds(t * C, C)]], sem_s[m])

    def _compute(m):
        eb, rb = ebuf[m], rows[m]

        def _row(i, _):
            i0 = i * 2
            for r in range(2):
                for jj in range(D // L):
                    sl = pl.ds(jj * L, L)
                    eb[i0 + r, sl] = jnp.maximum(rb[i0 + r, sl] + eb[i0 + r, sl], 0.0)
            return 0

        lax.fori_loop(0, C // 2, _row, 0)

    def _body(t, m, first):
        m1 = (m + 1) % 3
        m2 = (m + 2) % 3
        @pl.when(t < NK - 1)
        def _():
            _idx_cp(t + 1, m1).wait()
            _edge_cp(t + 1, m1).start()
            _gather_cp(m1).start()

        @pl.when(t < NK - 2)
        def _():
            _idx_cp(t + 2, m2).start()

        _edge_cp(t, m).wait()
        _gather_cp(m).wait()
        _compute(m)
        if not first:
            _scatter_cp(t - 1, m2).wait()
        _scatter_cp(t, m).start(add=True)

    pltpu.sync_copy(sidx_hbm.at[pl.ds(wid * EPT, C)], sidx[0])
    _edge_cp(0, 0).start()
    _gather_cp(0).start()
    _idx_cp(1, 1).start()
    _body(0, 0, True)

    def _trio(k, _):
        for j in range(3):
            t = 1 + 3 * k + j
            _body(t, (1 + j) % 3, False)
        return 0

    lax.fori_loop(0, (NK - 1) // 3, _trio, 0)
    _scatter_cp(NK - 1, (NK - 1) % 3).wait()

    plsc.subcore_barrier()
    r0 = s * ROWS_PER_SUB
    pltpu.sync_copy(acc.at[pl.ds(r0, ROWS_PER_SUB), :],
                    out_hbm.at[c, pl.ds(r0, ROWS_PER_SUB), :])


def _combine_body(eps_ref, x_ref, p0_ref, p1_ref, o_ref):
    o_ref[...] = ((1.0 + eps_ref[0]) * x_ref[...]
                  + p0_ref[0, ...] + p1_ref[0, ...])


_ROWS_BLK = 1000


def _combine(eps, x, partials):
    blk = pl.BlockSpec((_ROWS_BLK, D), lambda i: (i, 0))
    pblk0 = pl.BlockSpec((1, _ROWS_BLK, D), lambda i: (0, i, 0))
    pblk1 = pl.BlockSpec((1, _ROWS_BLK, D), lambda i: (1, i, 0))
    return pl.pallas_call(
        _combine_body,
        out_shape=jax.ShapeDtypeStruct((N_NODES, D), jnp.float32),
        grid=(N_NODES // _ROWS_BLK,),
        in_specs=[pl.BlockSpec(memory_space=pltpu.SMEM), blk, pblk0, pblk1],
        out_specs=blk,
    )(eps, x, partials, partials)


def kernel(node_feat, edge_index, edge_feat, eps):
    eidx = edge_index.astype(jnp.int32)
    partials = _sc_scatter(node_feat, eidx[0], eidx[1], edge_feat)
    return _combine(eps, node_feat, partials)


# speedup vs baseline: 1.0572x; 1.0572x over previous
import functools

import jax
import jax.numpy as jnp
from jax import lax
from jax.experimental import pallas as pl
from jax.experimental.pallas import tpu as pltpu
from jax.experimental.pallas import tpu_sc as plsc

N_NODES = 10000
N_PAD = 10240
N_EDGES = 320000
D = 128
L = 16

NC = 2
NS = 16
NW = NC * NS

EPT = N_EDGES // NW
C = 40
NK = EPT // C
ROWS_PER_SUB = N_PAD // NS
OUT_CHUNK = 32

_mesh = plsc.VectorSubcoreMesh(core_axis_name="c", subcore_axis_name="s")


@functools.partial(
    pl.kernel,
    out_type=jax.ShapeDtypeStruct((NC, N_PAD, D), jnp.float32),
    mesh=_mesh,
    scratch_types=[
        pltpu.VMEM_SHARED((N_PAD, D), jnp.float32),
        pltpu.VMEM((EPT,), jnp.int32),
        pltpu.VMEM((C,), jnp.int32),
        pltpu.VMEM((C,), jnp.int32),
        pltpu.VMEM((C,), jnp.int32),
        pltpu.VMEM((C, D), jnp.float32),
        pltpu.VMEM((C, D), jnp.float32),
        pltpu.VMEM((C, D), jnp.float32),
        pltpu.VMEM((C, D), jnp.float32),
        pltpu.VMEM((C, D), jnp.float32),
        pltpu.VMEM((C, D), jnp.float32),
        pltpu.VMEM((OUT_CHUNK, D), jnp.float32),
        pltpu.SemaphoreType.DMA,
        pltpu.SemaphoreType.DMA,
        pltpu.SemaphoreType.DMA,
        pltpu.SemaphoreType.DMA,
        pltpu.SemaphoreType.DMA,
        pltpu.SemaphoreType.DMA,
        pltpu.SemaphoreType.DMA,
        pltpu.SemaphoreType.DMA,
        pltpu.SemaphoreType.DMA,
        pltpu.SemaphoreType.DMA,
        pltpu.SemaphoreType.DMA,
        pltpu.SemaphoreType.DMA,
    ],
)
def _sc_scatter(node_hbm, eidx_hbm, edge_hbm, out_hbm,
                acc, didx, si0, si1, si2, eb0, eb1, eb2, rw0, rw1, rw2, obuf,
                smi0, smi1, smi2, sme0, sme1, sme2,
                smg0, smg1, smg2, sms0, sms1, sms2):
    c = lax.axis_index("c")
    s = lax.axis_index("s")
    wid = s * NC + c
    sidx = (si0, si1, si2)
    ebuf = (eb0, eb1, eb2)
    rows = (rw0, rw1, rw2)
    sem_i = (smi0, smi1, smi2)
    sem_e = (sme0, sme1, sme2)
    sem_g = (smg0, smg1, smg2)
    sem_s = (sms0, sms1, sms2)

    pltpu.sync_copy(eidx_hbm.at[pl.ds(N_EDGES + wid * EPT, EPT)], didx)

    zeros = jnp.zeros((L,), jnp.float32)

    def _zrow(i, _):
        for jj in range(D // L):
            obuf[i, pl.ds(jj * L, L)] = zeros
        return 0

    lax.fori_loop(0, OUT_CHUNK, _zrow, 0)
    zcps = [pltpu.make_async_copy(
        obuf, acc.at[pl.ds(s * ROWS_PER_SUB + p * OUT_CHUNK, OUT_CHUNK), :],
        smi0) for p in range(ROWS_PER_SUB // OUT_CHUNK)]
    for cp in zcps:
        cp.start()
    for cp in zcps:
        cp.wait()
    plsc.subcore_barrier()

    def _idx_cp(t, m):
        return pltpu.make_async_copy(
            eidx_hbm.at[pl.ds(wid * EPT + t * C, C)], sidx[m], sem_i[m])

    def _edge_cp(t, m):
        return pltpu.make_async_copy(
            edge_hbm.at[pl.ds(wid * EPT + t * C, C), :], ebuf[m], sem_e[m])

    def _gather_cp(m):
        return pltpu.make_async_copy(
            node_hbm.at[sidx[m]], rows[m], sem_g[m])

    def _scatter_cp(t, m):
        return pltpu.make_async_copy(
            ebuf[m], acc.at[didx.at[pl.ds(t * C, C)]], sem_s[m])

    def _compute(m):
        eb, rb = ebuf[m], rows[m]

        def _row(i, _):
            i0 = i * 2
            for r in range(2):
                for jj in range(D // L):
                    sl = pl.ds(jj * L, L)
                    eb[i0 + r, sl] = jnp.maximum(rb[i0 + r, sl] + eb[i0 + r, sl], 0.0)
            return 0

        lax.fori_loop(0, C // 2, _row, 0)

    def _body(t, m, first):
        m1 = (m + 1) % 3
        m2 = (m + 2) % 3
        @pl.when(t < NK - 1)
        def _():
            _idx_cp(t + 1, m1).wait()
            _edge_cp(t + 1, m1).start()
            _gather_cp(m1).start()

        @pl.when(t < NK - 2)
        def _():
            _idx_cp(t + 2, m2).start()

        _edge_cp(t, m).wait()
        _gather_cp(m).wait()
        _compute(m)
        if not first:
            _scatter_cp(t - 1, m2).wait()
        _scatter_cp(t, m).start(add=True)

    pltpu.sync_copy(eidx_hbm.at[pl.ds(wid * EPT, C)], sidx[0])
    _edge_cp(0, 0).start()
    _gather_cp(0).start()
    _idx_cp(1, 1).start()
    _body(0, 0, True)

    def _trio(k, _):
        for j in range(3):
            t = 1 + 3 * k + j
            _body(t, (1 + j) % 3, False)
        return 0

    lax.fori_loop(0, (NK - 1) // 3, _trio, 0)
    _scatter_cp(NK - 1, (NK - 1) % 3).wait()

    plsc.subcore_barrier()
    r0 = s * ROWS_PER_SUB
    pltpu.sync_copy(acc.at[pl.ds(r0, ROWS_PER_SUB), :],
                    out_hbm.at[c, pl.ds(r0, ROWS_PER_SUB), :])


def _combine_body(eps_ref, x_ref, p0_ref, p1_ref, o_ref):
    o_ref[...] = ((1.0 + eps_ref[0]) * x_ref[...]
                  + p0_ref[0, ...] + p1_ref[0, ...])


_ROWS_BLK = 2000


def _combine(eps, x, partials):
    blk = pl.BlockSpec((_ROWS_BLK, D), lambda i: (i, 0))
    pblk0 = pl.BlockSpec((1, _ROWS_BLK, D), lambda i: (0, i, 0))
    pblk1 = pl.BlockSpec((1, _ROWS_BLK, D), lambda i: (1, i, 0))
    return pl.pallas_call(
        _combine_body,
        out_shape=jax.ShapeDtypeStruct((N_NODES, D), jnp.float32),
        grid=(N_NODES // _ROWS_BLK,),
        in_specs=[pl.BlockSpec(memory_space=pltpu.SMEM), blk, pblk0, pblk1],
        out_specs=blk,
    )(eps, x, partials, partials)


def kernel(node_feat, edge_index, edge_feat, eps):
    eidx = edge_index.astype(jnp.int32).reshape(-1)
    partials = _sc_scatter(node_feat, eidx, edge_feat)
    return _combine(eps, node_feat, partials)


# speedup vs baseline: 1.0608x; 1.0034x over previous
import functools

import jax
import jax.numpy as jnp
from jax import lax
from jax.experimental import pallas as pl
from jax.experimental.pallas import tpu as pltpu
from jax.experimental.pallas import tpu_sc as plsc

N_NODES = 10000
N_PAD = 10240
N_EDGES = 320000
D = 128
L = 16

NC = 2
NS = 16
NW = NC * NS

EPT = N_EDGES // NW
C = 40
NK = EPT // C
ROWS_PER_SUB = N_PAD // NS
OUT_CHUNK = 32

_mesh = plsc.VectorSubcoreMesh(core_axis_name="c", subcore_axis_name="s")


@functools.partial(
    pl.kernel,
    out_type=jax.ShapeDtypeStruct((NC, N_PAD, D), jnp.float32),
    mesh=_mesh,
    scratch_types=[
        pltpu.VMEM_SHARED((N_PAD, D), jnp.float32),
        pltpu.VMEM((EPT,), jnp.int32),
        pltpu.VMEM((C,), jnp.int32),
        pltpu.VMEM((C,), jnp.int32),
        pltpu.VMEM((C,), jnp.int32),
        pltpu.VMEM((C, D), jnp.float32),
        pltpu.VMEM((C, D), jnp.float32),
        pltpu.VMEM((C, D), jnp.float32),
        pltpu.VMEM((C, D), jnp.float32),
        pltpu.VMEM((C, D), jnp.float32),
        pltpu.VMEM((C, D), jnp.float32),
        pltpu.VMEM((OUT_CHUNK, D), jnp.float32),
        pltpu.SemaphoreType.DMA,
        pltpu.SemaphoreType.DMA,
        pltpu.SemaphoreType.DMA,
        pltpu.SemaphoreType.DMA,
        pltpu.SemaphoreType.DMA,
        pltpu.SemaphoreType.DMA,
        pltpu.SemaphoreType.DMA,
        pltpu.SemaphoreType.DMA,
        pltpu.SemaphoreType.DMA,
        pltpu.SemaphoreType.DMA,
        pltpu.SemaphoreType.DMA,
        pltpu.SemaphoreType.DMA,
    ],
)
def _sc_scatter(node_hbm, eidx_hbm, edge_hbm, out_hbm,
                acc, didx, si0, si1, si2, eb0, eb1, eb2, rw0, rw1, rw2, obuf,
                smi0, smi1, smi2, sme0, sme1, sme2,
                smg0, smg1, smg2, sms0, sms1, sms2):
    c = lax.axis_index("c")
    s = lax.axis_index("s")
    wid = s * NC + c
    sidx = (si0, si1, si2)
    ebuf = (eb0, eb1, eb2)
    rows = (rw0, rw1, rw2)
    sem_i = (smi0, smi1, smi2)
    sem_e = (sme0, sme1, sme2)
    sem_g = (smg0, smg1, smg2)
    sem_s = (sms0, sms1, sms2)

    didx_cp = pltpu.make_async_copy(
        eidx_hbm.at[pl.ds(N_EDGES + wid * EPT, EPT)], didx, sms0)
    didx_cp.start()

    zeros = jnp.zeros((L,), jnp.float32)

    def _zrow(i, _):
        for jj in range(D // L):
            obuf[i, pl.ds(jj * L, L)] = zeros
        return 0

    lax.fori_loop(0, OUT_CHUNK, _zrow, 0)
    zcps = [pltpu.make_async_copy(
        obuf, acc.at[pl.ds(s * ROWS_PER_SUB + p * OUT_CHUNK, OUT_CHUNK), :],
        smi0) for p in range(ROWS_PER_SUB // OUT_CHUNK)]
    for cp in zcps:
        cp.start()
    for cp in zcps:
        cp.wait()
    didx_cp.wait()
    plsc.subcore_barrier()

    def _idx_cp(t, m):
        return pltpu.make_async_copy(
            eidx_hbm.at[pl.ds(wid * EPT + t * C, C)], sidx[m], sem_i[m])

    def _edge_cp(t, m):
        return pltpu.make_async_copy(
            edge_hbm.at[pl.ds(wid * EPT + t * C, C), :], ebuf[m], sem_e[m])

    def _gather_cp(m):
        return pltpu.make_async_copy(
            node_hbm.at[sidx[m]], rows[m], sem_g[m])

    def _scatter_cp(t, m):
        return pltpu.make_async_copy(
            ebuf[m], acc.at[didx.at[pl.ds(t * C, C)]], sem_s[m])

    def _compute(m):
        eb, rb = ebuf[m], rows[m]

        def _row(i, _):
            i0 = i * 4
            for r in range(4):
                for jj in range(D // L):
                    sl = pl.ds(jj * L, L)
                    eb[i0 + r, sl] = jnp.maximum(rb[i0 + r, sl] + eb[i0 + r, sl], 0.0)
            return 0

        lax.fori_loop(0, C // 4, _row, 0)

    def _body(t, m, first):
        m1 = (m + 1) % 3
        m2 = (m + 2) % 3
        @pl.when(t < NK - 1)
        def _():
            _idx_cp(t + 1, m1).wait()
            _edge_cp(t + 1, m1).start()
            _gather_cp(m1).start()

        @pl.when(t < NK - 2)
        def _():
            _idx_cp(t + 2, m2).start()

        _edge_cp(t, m).wait()
        _gather_cp(m).wait()
        _compute(m)
        if not first:
            _scatter_cp(t - 1, m2).wait()
        _scatter_cp(t, m).start(add=True)

    pltpu.sync_copy(eidx_hbm.at[pl.ds(wid * EPT, C)], sidx[0])
    _edge_cp(0, 0).start()
    _gather_cp(0).start()
    _idx_cp(1, 1).start()
    _body(0, 0, True)

    def _trio(k, _):
        for j in range(3):
            t = 1 + 3 * k + j
            _body(t, (1 + j) % 3, False)
        return 0

    lax.fori_loop(0, (NK - 1) // 3, _trio, 0)
    _scatter_cp(NK - 1, (NK - 1) % 3).wait()

    plsc.subcore_barrier()
    r0 = s * ROWS_PER_SUB
    pltpu.sync_copy(acc.at[pl.ds(r0, ROWS_PER_SUB), :],
                    out_hbm.at[c, pl.ds(r0, ROWS_PER_SUB), :])


def _combine_body(eps_ref, x_ref, p0_ref, p1_ref, o_ref):
    o_ref[...] = ((1.0 + eps_ref[0]) * x_ref[...]
                  + p0_ref[0, ...] + p1_ref[0, ...])


_ROWS_BLK = 2000


def _combine(eps, x, partials):
    blk = pl.BlockSpec((_ROWS_BLK, D), lambda i: (i, 0))
    pblk0 = pl.BlockSpec((1, _ROWS_BLK, D), lambda i: (0, i, 0))
    pblk1 = pl.BlockSpec((1, _ROWS_BLK, D), lambda i: (1, i, 0))
    return pl.pallas_call(
        _combine_body,
        out_shape=jax.ShapeDtypeStruct((N_NODES, D), jnp.float32),
        grid=(N_NODES // _ROWS_BLK,),
        in_specs=[pl.BlockSpec(memory_space=pltpu.SMEM), blk, pblk0, pblk1],
        out_specs=blk,
    )(eps, x, partials, partials)


def kernel(node_feat, edge_index, edge_feat, eps):
    eidx = edge_index.astype(jnp.int32).reshape(-1)
    partials = _sc_scatter(node_feat, eidx, edge_feat)
    return _combine(eps, node_feat, partials)
